# baseline (speedup 1.0000x reference)
import functools

import jax
import jax.numpy as jnp
from jax import lax
from jax.experimental import pallas as pl
from jax.experimental.pallas import tpu as pltpu
from jax.experimental.pallas import tpu_sc as plsc

N_NODES = 10000
N_EDGES = 320000
DIM = 128
NUM_FEATURES = 128
NUM_CLASSES = 10
TEMPERATURE = 0.6

NC = 2
NS = 16
NW = NC * NS
EDGES_PER_TILE = N_EDGES // NW
CHUNK = 128
NITER = EDGES_PER_TILE // CHUNK
TAIL = EDGES_PER_TILE - NITER * CHUNK

NP_PAD = 10240
DEG_ZROWS = NP_PAD // NS
ACC_ROWS = NP_PAD // NS
ZB_ROWS = 32
WB_ROWS = 128

_MESH = plsc.VectorSubcoreMesh(core_axis_name="c", subcore_axis_name="s")


@functools.partial(
    pl.kernel,
    mesh=_MESH,
    out_type=jax.ShapeDtypeStruct((NC, NP_PAD), jnp.float32),
    scratch_types=[
        pltpu.VMEM((4, CHUNK), jnp.int32),
        pltpu.VMEM((CHUNK,), jnp.float32),
        pltpu.VMEM((TAIL,), jnp.int32),
        pltpu.VMEM((TAIL,), jnp.float32),
        pltpu.VMEM((DEG_ZROWS,), jnp.float32),
        pltpu.VMEM_SHARED((NP_PAD,), jnp.float32),
        pltpu.SemaphoreType.DMA,
        pltpu.SemaphoreType.DMA,
    ],
)
def _deg_kernel(dst_hbm, out_hbm, dst_v, ones_v, dst_t, ones_t, zb_v, acc_sp,
                sem_d, sem_sc):
    c = lax.axis_index("c")
    s = lax.axis_index("s")
    wid = s * NC + c

    def fill_zeros(i, carry):
        zb_v[pl.ds(i * 16, 16)] = jnp.zeros((16,), jnp.float32)
        return carry

    lax.fori_loop(0, DEG_ZROWS // 16, fill_zeros, 0)

    def fill_ones(i, carry):
        ones_v[pl.ds(i * 16, 16)] = jnp.ones((16,), jnp.float32)
        return carry

    lax.fori_loop(0, CHUNK // 16, fill_ones, 0)
    ones_t[...] = jnp.ones((TAIL,), jnp.float32)

    pltpu.sync_copy(zb_v, acc_sp.at[pl.ds(s * DEG_ZROWS, DEG_ZROWS)])
    plsc.subcore_barrier()

    def ebase(i):
        return pl.multiple_of(wid * EDGES_PER_TILE + i * CHUNK, 8)

    def issue_idx(i, p):
        pltpu.async_copy(dst_hbm.at[pl.ds(ebase(i), CHUNK)], dst_v.at[p],
                         sem_d)

    def wait_idx(i, p):
        pltpu.make_async_copy(dst_hbm.at[pl.ds(ebase(i), CHUNK)],
                              dst_v.at[p], sem_d).wait()

    def wait_sc(p):
        pltpu.make_async_copy(ones_v, acc_sp.at[dst_v.at[p]], sem_sc).wait()

    issue_idx(0, 0)
    issue_idx(1, 1)

    def body(i, carry):
        p = lax.rem(i, 4)
        wait_idx(i, p)

        @pl.when(i >= 2)
        def _():
            wait_sc(lax.rem(i - 2, 4))

        pltpu.async_copy(ones_v, acc_sp.at[dst_v.at[p]], sem_sc, add=True)

        @pl.when(i + 2 < NITER)
        def _():
            issue_idx(i + 2, lax.rem(i + 2, 4))

        return carry

    lax.fori_loop(0, NITER, body, 0)
    wait_sc(lax.rem(NITER - 2, 4))
    wait_sc(lax.rem(NITER - 1, 4))
    tbase = pl.multiple_of(wid * EDGES_PER_TILE + NITER * CHUNK, 8)
    pltpu.sync_copy(dst_hbm.at[pl.ds(tbase, TAIL)], dst_t)
    pltpu.sync_copy(ones_t, acc_sp.at[dst_t], add=True)
    plsc.subcore_barrier()
    pltpu.sync_copy(
        acc_sp.at[pl.ds(s * DEG_ZROWS, DEG_ZROWS)],
        out_hbm.at[c, pl.ds(s * DEG_ZROWS, DEG_ZROWS)],
    )


@functools.partial(
    pl.kernel,
    mesh=_MESH,
    out_type=jax.ShapeDtypeStruct((NC, NP_PAD, DIM), jnp.float32),
    scratch_types=[
        pltpu.VMEM((2, CHUNK), jnp.int32),
        pltpu.VMEM((4, CHUNK), jnp.int32),
        pltpu.VMEM((2, CHUNK, DIM), jnp.float32),
        pltpu.VMEM((TAIL,), jnp.int32),
        pltpu.VMEM((TAIL,), jnp.int32),
        pltpu.VMEM((TAIL, DIM), jnp.float32),
        pltpu.VMEM((ZB_ROWS, DIM), jnp.float32),
        pltpu.VMEM_SHARED((NP_PAD, DIM), jnp.float32),
        pltpu.SemaphoreType.DMA,
        pltpu.SemaphoreType.DMA,
        pltpu.SemaphoreType.DMA,
        pltpu.SemaphoreType.DMA,
        pltpu.SemaphoreType.DMA,
    ],
)
def _scatter_kernel(hp_hbm, src_hbm, dst_hbm, out_hbm, src_v, dst_v, rows_v,
                    src_t, dst_t, rows_t, zb_v, acc_sp, sem_s, sem_d, sem_g,
                    sem_sc, sem_z):
    c = lax.axis_index("c")
    s = lax.axis_index("s")
    wid = s * NC + c

    def fill_zeros(i, carry):
        for j in range(DIM // 16):
            zb_v[i, pl.ds(j * 16, 16)] = jnp.zeros((16,), jnp.float32)
        return carry

    lax.fori_loop(0, ZB_ROWS, fill_zeros, 0)
    for j in range(ACC_ROWS // ZB_ROWS):
        pltpu.async_copy(
            zb_v, acc_sp.at[pl.ds(s * ACC_ROWS + j * ZB_ROWS, ZB_ROWS), :],
            sem_z)

    def ebase(i):
        return pl.multiple_of(wid * EDGES_PER_TILE + i * CHUNK, 8)

    def issue_idx(i, p2, p4):
        pltpu.async_copy(src_hbm.at[pl.ds(ebase(i), CHUNK)], src_v.at[p2],
                         sem_s)
        pltpu.async_copy(dst_hbm.at[pl.ds(ebase(i), CHUNK)], dst_v.at[p4],
                         sem_d)

    def wait_src(i, p2):
        pltpu.make_async_copy(src_hbm.at[pl.ds(ebase(i), CHUNK)],
                              src_v.at[p2], sem_s).wait()

    def wait_dst(i, p4):
        pltpu.make_async_copy(dst_hbm.at[pl.ds(ebase(i), CHUNK)],
                              dst_v.at[p4], sem_d).wait()

    def wait_sc(p2, p4):
        pltpu.make_async_copy(rows_v.at[p2], acc_sp.at[dst_v.at[p4]],
                              sem_sc).wait()

    issue_idx(0, 0, 0)
    issue_idx(1, 1, 1)
    wait_src(0, 0)
    pltpu.async_copy(hp_hbm.at[src_v.at[0]], rows_v.at[0], sem_g)
    for j in range(ACC_ROWS // ZB_ROWS):
        pltpu.make_async_copy(
            zb_v, acc_sp.at[pl.ds(s * ACC_ROWS + j * ZB_ROWS, ZB_ROWS), :],
            sem_z).wait()
    plsc.subcore_barrier()

    def body(i, carry):
        p2 = lax.rem(i, 2)
        q2 = 1 - p2
        p4 = lax.rem(i, 4)
        pltpu.make_async_copy(hp_hbm.at[src_v.at[p2]], rows_v.at[p2],
                              sem_g).wait()

        @pl.when(i >= 1)
        def _():
            wait_sc(q2, lax.rem(i - 1, 4))

        @pl.when(i + 1 < NITER)
        def _():
            wait_src(i + 1, q2)
            pltpu.async_copy(hp_hbm.at[src_v.at[q2]], rows_v.at[q2], sem_g)

        wait_dst(i, p4)
        pltpu.async_copy(rows_v.at[p2], acc_sp.at[dst_v.at[p4]], sem_sc,
                         add=True)

        @pl.when(i + 2 < NITER)
        def _():
            issue_idx(i + 2, p2, lax.rem(i + 2, 4))

        return carry

    lax.fori_loop(0, NITER, body, 0)
    wait_sc(lax.rem(NITER - 1, 2), lax.rem(NITER - 1, 4))
    tbase = pl.multiple_of(wid * EDGES_PER_TILE + NITER * CHUNK, 8)
    pltpu.sync_copy(src_hbm.at[pl.ds(tbase, TAIL)], src_t)
    pltpu.sync_copy(dst_hbm.at[pl.ds(tbase, TAIL)], dst_t)
    pltpu.async_copy(hp_hbm.at[src_t], rows_t, sem_g).wait()
    pltpu.sync_copy(rows_t, acc_sp.at[dst_t], add=True)
    plsc.subcore_barrier()
    for j in range(ACC_ROWS // WB_ROWS):
        rbase = s * ACC_ROWS + j * WB_ROWS
        pltpu.sync_copy(
            acc_sp.at[pl.ds(rbase, WB_ROWS), :],
            out_hbm.at[c, pl.ds(rbase, WB_ROWS), :],
        )


NBLK = 10
BLK = N_NODES // NBLK


def _mm0_body(deg_ref, x_ref, w_ref, hp_ref, dinv_ref):
    deg = deg_ref[:, 0] + deg_ref[:, 1] + 1.0
    dinv = jnp.where(deg > 0, lax.rsqrt(deg), 0.0).reshape(BLK, 1)
    dinv_ref[...] = dinv
    hp_ref[...] = jnp.dot(
        x_ref[...], w_ref[...], preferred_element_type=jnp.float32) * dinv


def _mid_body(a_ref, hp_ref, dinv_ref, b_ref, w_ref, out_ref):
    dinv = dinv_ref[...]
    z = (a_ref[0] + a_ref[1] + hp_ref[...]) * dinv + b_ref[...]
    u = jnp.where(z > 0, z, 0.01 * z)
    out_ref[...] = jnp.dot(
        u, w_ref[...], preferred_element_type=jnp.float32) * dinv


def _mm0_call(deg2, x, w):
    return pl.pallas_call(
        _mm0_body,
        grid=(NBLK,),
        in_specs=[
            pl.BlockSpec((BLK, 2), lambda i: (i, 0)),
            pl.BlockSpec((BLK, NUM_FEATURES), lambda i: (i, 0)),
            pl.BlockSpec((NUM_FEATURES, DIM), lambda i: (0, 0)),
        ],
        out_specs=(
            pl.BlockSpec((BLK, DIM), lambda i: (i, 0)),
            pl.BlockSpec((BLK, 1), lambda i: (i, 0)),
        ),
        out_shape=(
            jax.ShapeDtypeStruct((N_NODES, DIM), jnp.float32),
            jax.ShapeDtypeStruct((N_NODES, 1), jnp.float32),
        ),
    )(deg2, x, w)


def _mid_call(a, hp, dinv, b2, w):
    return pl.pallas_call(
        _mid_body,
        grid=(NBLK,),
        in_specs=[
            pl.BlockSpec((NC, BLK, DIM), lambda i: (0, i, 0)),
            pl.BlockSpec((BLK, DIM), lambda i: (i, 0)),
            pl.BlockSpec((BLK, 1), lambda i: (i, 0)),
            pl.BlockSpec((1, DIM), lambda i: (0, 0)),
            pl.BlockSpec((DIM, DIM), lambda i: (0, 0)),
        ],
        out_specs=pl.BlockSpec((BLK, DIM), lambda i: (i, 0)),
        out_shape=jax.ShapeDtypeStruct((N_NODES, DIM), jnp.float32),
    )(a, hp, dinv, b2, w)


def _final_body(a_ref, hp_ref, dinv_ref, b_ref, lw_ref, lb_ref,
                conc_ref, lp_ref):
    a = a_ref[0, :N_NODES] + a_ref[1, :N_NODES]
    z = (a + hp_ref[...]) * dinv_ref[...] + b_ref[...]
    u = jnp.where(z > 0, z, 0.01 * z)
    sm = jnp.exp(u - jnp.max(u, axis=-1, keepdims=True))
    conc_ref[...] = sm
    w0 = lw_ref[...]
    gamma = jnp.abs(w0) / TEMPERATURE
    ex = jnp.exp(gamma - jnp.max(gamma, axis=1, keepdims=True))
    alpha = ex / jnp.sum(ex, axis=1, keepdims=True)
    alpha_norm = alpha / jnp.max(alpha, axis=1, keepdims=True)
    veff = (alpha_norm * w0).T
    y = jnp.dot(sm, veff, preferred_element_type=jnp.float32) + lb_ref[...]
    m = jnp.max(y, axis=0, keepdims=True)
    lse = m + jnp.log(jnp.sum(jnp.exp(y - m), axis=0, keepdims=True))
    lp_ref[...] = y - lse


def _tc(body, out_shape, *args):
    return pl.pallas_call(body, out_shape=out_shape)(*args)


def kernel(x, edge_index, W0, b0, W1, b1, W2, b2, W3, b3, lens_w, lens_b):
    f32 = jnp.float32
    src = edge_index[0]
    dst = edge_index[1]
    deg2 = _deg_kernel(dst)[:, :N_NODES].T
    hp, dinv = _mm0_call(deg2, x, W0)
    for b, w in ((b0, W1), (b1, W2), (b2, W3)):
        a = _scatter_kernel(hp, src, dst)
        hp = _mid_call(a, hp, dinv, b.reshape(1, DIM), w)
    a = _scatter_kernel(hp, src, dst)
    concepts, log_probs = _tc(
        _final_body,
        (jax.ShapeDtypeStruct((N_NODES, DIM), f32),
         jax.ShapeDtypeStruct((N_NODES, NUM_CLASSES), f32)),
        a, hp, dinv, b3.reshape(1, DIM),
        lens_w.reshape(NUM_CLASSES, DIM),
        lens_b.reshape(1, NUM_CLASSES))
    return (concepts, log_probs)


# speedup vs baseline: 1.0277x; 1.0277x over previous
import functools

import jax
import jax.numpy as jnp
from jax import lax
from jax.experimental import pallas as pl
from jax.experimental.pallas import tpu as pltpu
from jax.experimental.pallas import tpu_sc as plsc

N_NODES = 10000
N_EDGES = 320000
DIM = 128
NUM_FEATURES = 128
NUM_CLASSES = 10
TEMPERATURE = 0.6

NC = 2
NS = 16
NW = NC * NS
EDGES_PER_TILE = N_EDGES // NW
CHUNK = 128
NITER = EDGES_PER_TILE // CHUNK
TAIL = EDGES_PER_TILE - NITER * CHUNK

NP_PAD = 10240
DEG_ZROWS = NP_PAD // NS
ACC_ROWS = NP_PAD // NS
ZB_ROWS = 32
WB_ROWS = 128

_MESH = plsc.VectorSubcoreMesh(core_axis_name="c", subcore_axis_name="s")


@functools.partial(
    pl.kernel,
    mesh=_MESH,
    out_type=jax.ShapeDtypeStruct((NC, NP_PAD), jnp.float32),
    scratch_types=[
        pltpu.VMEM((4, CHUNK), jnp.int32),
        pltpu.VMEM((CHUNK,), jnp.float32),
        pltpu.VMEM((TAIL,), jnp.int32),
        pltpu.VMEM((TAIL,), jnp.float32),
        pltpu.VMEM((DEG_ZROWS,), jnp.float32),
        pltpu.VMEM_SHARED((NP_PAD,), jnp.float32),
        pltpu.SemaphoreType.DMA,
        pltpu.SemaphoreType.DMA,
    ],
)
def _deg_kernel(dst_hbm, out_hbm, dst_v, ones_v, dst_t, ones_t, zb_v, acc_sp,
                sem_d, sem_sc):
    c = lax.axis_index("c")
    s = lax.axis_index("s")
    wid = s * NC + c

    def fill_zeros(i, carry):
        zb_v[pl.ds(i * 16, 16)] = jnp.zeros((16,), jnp.float32)
        return carry

    lax.fori_loop(0, DEG_ZROWS // 16, fill_zeros, 0)

    def fill_ones(i, carry):
        ones_v[pl.ds(i * 16, 16)] = jnp.ones((16,), jnp.float32)
        return carry

    lax.fori_loop(0, CHUNK // 16, fill_ones, 0)
    ones_t[...] = jnp.ones((TAIL,), jnp.float32)

    pltpu.sync_copy(zb_v, acc_sp.at[pl.ds(s * DEG_ZROWS, DEG_ZROWS)])
    plsc.subcore_barrier()

    def ebase(i):
        return pl.multiple_of(wid * EDGES_PER_TILE + i * CHUNK, 8)

    def issue_idx(i, p):
        pltpu.async_copy(dst_hbm.at[pl.ds(ebase(i), CHUNK)], dst_v.at[p],
                         sem_d)

    def wait_idx(i, p):
        pltpu.make_async_copy(dst_hbm.at[pl.ds(ebase(i), CHUNK)],
                              dst_v.at[p], sem_d).wait()

    def wait_sc(p):
        pltpu.make_async_copy(ones_v, acc_sp.at[dst_v.at[p]], sem_sc).wait()

    issue_idx(0, 0)
    issue_idx(1, 1)

    def body(i, carry):
        p = lax.rem(i, 4)
        wait_idx(i, p)

        @pl.when(i >= 2)
        def _():
            wait_sc(lax.rem(i - 2, 4))

        pltpu.async_copy(ones_v, acc_sp.at[dst_v.at[p]], sem_sc, add=True)

        @pl.when(i + 2 < NITER)
        def _():
            issue_idx(i + 2, lax.rem(i + 2, 4))

        return carry

    lax.fori_loop(0, NITER, body, 0)
    wait_sc(lax.rem(NITER - 2, 4))
    wait_sc(lax.rem(NITER - 1, 4))
    tbase = pl.multiple_of(wid * EDGES_PER_TILE + NITER * CHUNK, 8)
    pltpu.sync_copy(dst_hbm.at[pl.ds(tbase, TAIL)], dst_t)
    pltpu.sync_copy(ones_t, acc_sp.at[dst_t], add=True)
    plsc.subcore_barrier()
    pltpu.sync_copy(
        acc_sp.at[pl.ds(s * DEG_ZROWS, DEG_ZROWS)],
        out_hbm.at[c, pl.ds(s * DEG_ZROWS, DEG_ZROWS)],
    )


@functools.partial(
    pl.kernel,
    mesh=_MESH,
    out_type=jax.ShapeDtypeStruct((NC, NP_PAD, DIM), jnp.float32),
    scratch_types=[
        pltpu.VMEM((2, CHUNK), jnp.int32),
        pltpu.VMEM((4, CHUNK), jnp.int32),
        pltpu.VMEM((2, CHUNK, DIM), jnp.float32),
        pltpu.VMEM((TAIL,), jnp.int32),
        pltpu.VMEM((TAIL,), jnp.int32),
        pltpu.VMEM((TAIL, DIM), jnp.float32),
        pltpu.VMEM((ZB_ROWS, DIM), jnp.float32),
        pltpu.VMEM_SHARED((NP_PAD, DIM), jnp.float32),
        pltpu.SemaphoreType.DMA,
        pltpu.SemaphoreType.DMA,
        pltpu.SemaphoreType.DMA,
        pltpu.SemaphoreType.DMA,
        pltpu.SemaphoreType.DMA,
    ],
)
def _scatter_kernel(hp_hbm, src_hbm, dst_hbm, out_hbm, src_v, dst_v, rows_v,
                    src_t, dst_t, rows_t, zb_v, acc_sp, sem_s, sem_d, sem_g,
                    sem_sc, sem_z):
    c = lax.axis_index("c")
    s = lax.axis_index("s")
    wid = s * NC + c

    def fill_zeros(i, carry):
        for j in range(DIM // 16):
            zb_v[i, pl.ds(j * 16, 16)] = jnp.zeros((16,), jnp.float32)
        return carry

    lax.fori_loop(0, ZB_ROWS, fill_zeros, 0)
    for j in range(ACC_ROWS // ZB_ROWS):
        pltpu.async_copy(
            zb_v, acc_sp.at[pl.ds(s * ACC_ROWS + j * ZB_ROWS, ZB_ROWS), :],
            sem_z)

    def ebase(i):
        return pl.multiple_of(wid * EDGES_PER_TILE + i * CHUNK, 8)

    def issue_idx(i, p2, p4):
        pltpu.async_copy(src_hbm.at[pl.ds(ebase(i), CHUNK)], src_v.at[p2],
                         sem_s)
        pltpu.async_copy(dst_hbm.at[pl.ds(ebase(i), CHUNK)], dst_v.at[p4],
                         sem_d)

    def wait_src(i, p2):
        pltpu.make_async_copy(src_hbm.at[pl.ds(ebase(i), CHUNK)],
                              src_v.at[p2], sem_s).wait()

    def wait_dst(i, p4):
        pltpu.make_async_copy(dst_hbm.at[pl.ds(ebase(i), CHUNK)],
                              dst_v.at[p4], sem_d).wait()

    def wait_sc(p2, p4):
        pltpu.make_async_copy(rows_v.at[p2], acc_sp.at[dst_v.at[p4]],
                              sem_sc).wait()

    issue_idx(0, 0, 0)
    issue_idx(1, 1, 1)
    wait_src(0, 0)
    pltpu.async_copy(hp_hbm.at[src_v.at[0]], rows_v.at[0], sem_g)
    for j in range(ACC_ROWS // ZB_ROWS):
        pltpu.make_async_copy(
            zb_v, acc_sp.at[pl.ds(s * ACC_ROWS + j * ZB_ROWS, ZB_ROWS), :],
            sem_z).wait()
    plsc.subcore_barrier()

    def body(i, carry):
        p2 = lax.rem(i, 2)
        q2 = 1 - p2
        p4 = lax.rem(i, 4)
        pltpu.make_async_copy(hp_hbm.at[src_v.at[p2]], rows_v.at[p2],
                              sem_g).wait()

        @pl.when(i >= 1)
        def _():
            wait_sc(q2, lax.rem(i - 1, 4))

        @pl.when(i + 1 < NITER)
        def _():
            wait_src(i + 1, q2)
            pltpu.async_copy(hp_hbm.at[src_v.at[q2]], rows_v.at[q2], sem_g)

        wait_dst(i, p4)
        pltpu.async_copy(rows_v.at[p2], acc_sp.at[dst_v.at[p4]], sem_sc,
                         add=True)

        @pl.when(i + 2 < NITER)
        def _():
            issue_idx(i + 2, p2, lax.rem(i + 2, 4))

        return carry

    lax.fori_loop(0, NITER, body, 0)
    wait_sc(lax.rem(NITER - 1, 2), lax.rem(NITER - 1, 4))
    tbase = pl.multiple_of(wid * EDGES_PER_TILE + NITER * CHUNK, 8)
    pltpu.sync_copy(src_hbm.at[pl.ds(tbase, TAIL)], src_t)
    pltpu.sync_copy(dst_hbm.at[pl.ds(tbase, TAIL)], dst_t)
    pltpu.async_copy(hp_hbm.at[src_t], rows_t, sem_g).wait()
    pltpu.sync_copy(rows_t, acc_sp.at[dst_t], add=True)
    plsc.subcore_barrier()
    for j in range(ACC_ROWS // WB_ROWS):
        rbase = s * ACC_ROWS + j * WB_ROWS
        pltpu.sync_copy(
            acc_sp.at[pl.ds(rbase, WB_ROWS), :],
            out_hbm.at[c, pl.ds(rbase, WB_ROWS), :],
        )


NBLK = 10
BLK = N_NODES // NBLK


def _mm0_body(deg_ref, x_ref, w_ref, hp_ref, dinv_ref):
    deg = deg_ref[:, 0] + deg_ref[:, 1] + 1.0
    dinv = jnp.where(deg > 0, lax.rsqrt(deg), 0.0).reshape(BLK, 1)
    dinv_ref[...] = dinv
    hp_ref[...] = jnp.dot(
        x_ref[...], w_ref[...], preferred_element_type=jnp.float32) * dinv


def _mid_body(a_ref, hp_ref, dinv_ref, b_ref, w_ref, out_ref):
    dinv = dinv_ref[...]
    z = (a_ref[0] + a_ref[1] + hp_ref[...]) * dinv + b_ref[...]
    u = jnp.where(z > 0, z, 0.01 * z)
    out_ref[...] = jnp.dot(
        u, w_ref[...], preferred_element_type=jnp.float32) * dinv


def _mm0s_body(deg_ref, x_ref, w_ref, hp_ref, dinv_ref):
    deg = deg_ref[0, :N_NODES] + deg_ref[1, :N_NODES] + 1.0
    dinv = jnp.where(deg > 0, lax.rsqrt(deg), 0.0).reshape(N_NODES, 1)
    dinv_ref[...] = dinv
    hp_ref[...] = jnp.dot(
        x_ref[...], w_ref[...], preferred_element_type=jnp.float32) * dinv


def _mids_body(a_ref, hp_ref, dinv_ref, b_ref, w_ref, out_ref):
    dinv = dinv_ref[...]
    a = a_ref[0, :N_NODES] + a_ref[1, :N_NODES]
    z = (a + hp_ref[...]) * dinv + b_ref[...]
    u = jnp.where(z > 0, z, 0.01 * z)
    out_ref[...] = jnp.dot(
        u, w_ref[...], preferred_element_type=jnp.float32) * dinv


def _mm0_call(deg2, x, w):
    return pl.pallas_call(
        _mm0_body,
        grid=(NBLK,),
        in_specs=[
            pl.BlockSpec((BLK, 2), lambda i: (i, 0)),
            pl.BlockSpec((BLK, NUM_FEATURES), lambda i: (i, 0)),
            pl.BlockSpec((NUM_FEATURES, DIM), lambda i: (0, 0)),
        ],
        out_specs=(
            pl.BlockSpec((BLK, DIM), lambda i: (i, 0)),
            pl.BlockSpec((BLK, 1), lambda i: (i, 0)),
        ),
        out_shape=(
            jax.ShapeDtypeStruct((N_NODES, DIM), jnp.float32),
            jax.ShapeDtypeStruct((N_NODES, 1), jnp.float32),
        ),
    )(deg2, x, w)


def _mid_call(a, hp, dinv, b2, w):
    return pl.pallas_call(
        _mid_body,
        grid=(NBLK,),
        in_specs=[
            pl.BlockSpec((NC, BLK, DIM), lambda i: (0, i, 0)),
            pl.BlockSpec((BLK, DIM), lambda i: (i, 0)),
            pl.BlockSpec((BLK, 1), lambda i: (i, 0)),
            pl.BlockSpec((1, DIM), lambda i: (0, 0)),
            pl.BlockSpec((DIM, DIM), lambda i: (0, 0)),
        ],
        out_specs=pl.BlockSpec((BLK, DIM), lambda i: (i, 0)),
        out_shape=jax.ShapeDtypeStruct((N_NODES, DIM), jnp.float32),
    )(a, hp, dinv, b2, w)


def _final_body(a_ref, hp_ref, dinv_ref, b_ref, lw_ref, lb_ref,
                conc_ref, lp_ref):
    a = a_ref[0, :N_NODES] + a_ref[1, :N_NODES]
    z = (a + hp_ref[...]) * dinv_ref[...] + b_ref[...]
    u = jnp.where(z > 0, z, 0.01 * z)
    sm = jnp.exp(u - jnp.max(u, axis=-1, keepdims=True))
    conc_ref[...] = sm
    w0 = lw_ref[...]
    gamma = jnp.abs(w0) / TEMPERATURE
    ex = jnp.exp(gamma - jnp.max(gamma, axis=1, keepdims=True))
    alpha = ex / jnp.sum(ex, axis=1, keepdims=True)
    alpha_norm = alpha / jnp.max(alpha, axis=1, keepdims=True)
    veff = (alpha_norm * w0).T
    y = jnp.dot(sm, veff, preferred_element_type=jnp.float32) + lb_ref[...]
    m = jnp.max(y, axis=0, keepdims=True)
    lse = m + jnp.log(jnp.sum(jnp.exp(y - m), axis=0, keepdims=True))
    lp_ref[...] = y - lse


def _tc(body, out_shape, *args):
    return pl.pallas_call(body, out_shape=out_shape)(*args)


def kernel(x, edge_index, W0, b0, W1, b1, W2, b2, W3, b3, lens_w, lens_b):
    f32 = jnp.float32
    src = edge_index[0]
    dst = edge_index[1]
    deg2 = _deg_kernel(dst)
    hp, dinv = _tc(
        _mm0s_body,
        (jax.ShapeDtypeStruct((N_NODES, DIM), f32),
         jax.ShapeDtypeStruct((N_NODES, 1), f32)),
        deg2, x, W0)
    mid_shape = jax.ShapeDtypeStruct((N_NODES, DIM), f32)
    for b, w in ((b0, W1), (b1, W2), (b2, W3)):
        a = _scatter_kernel(hp, src, dst)
        hp = _tc(_mids_body, mid_shape, a, hp, dinv, b.reshape(1, DIM), w)
    a = _scatter_kernel(hp, src, dst)
    concepts, log_probs = _tc(
        _final_body,
        (jax.ShapeDtypeStruct((N_NODES, DIM), f32),
         jax.ShapeDtypeStruct((N_NODES, NUM_CLASSES), f32)),
        a, hp, dinv, b3.reshape(1, DIM),
        lens_w.reshape(NUM_CLASSES, DIM),
        lens_b.reshape(1, NUM_CLASSES))
    return (concepts, log_probs)
